# baseline (speedup 1.0000x reference)
import jax
import jax.numpy as jnp
from jax import lax
from jax.experimental import pallas as pl
from jax.experimental.pallas import tpu as pltpu
from jax.experimental.pallas import tpu_sc as plsc

_TOPK = 25
_BLK = 2048
_NSEL_PAD = 256
_NTILES = 16
_SEL_PER_TILE = _NSEL_PAD // _NTILES


def _score_kernel(v_ref, nf_ref, e_ref, et_ref):
    v = v_ref[...]
    qn = v / jnp.maximum(jnp.sqrt(jnp.sum(v * v, axis=1, keepdims=True)), 1e-12)
    k = nf_ref[...]
    kn = k / jnp.maximum(jnp.sqrt(jnp.sum(k * k, axis=1, keepdims=True)), 1e-12)
    s = jax.lax.dot_general(qn, kn, (((1,), (1,)), ((), ())),
                            preferred_element_type=jnp.float32)
    e_ref[...] = jnp.exp(jnp.clip(s, 0.0, 1.0))
    st = jax.lax.dot_general(kn, qn, (((1,), (1,)), ((), ())),
                             preferred_element_type=jnp.float32)
    et_ref[...] = jnp.exp(jnp.clip(st, 0.0, 1.0))


def _topk_kernel(e_ref, tv_ref, idx_ref, sc_ref, work_ref):
    work_ref[...] = e_ref[...]
    nb, n = work_ref.shape
    gate = 2.0 * jax.nn.sigmoid(tv_ref[0])
    den = jnp.sum(e_ref[...], axis=1, keepdims=True)
    sc_ref[...] = gate / den
    iota = jax.lax.broadcasted_iota(jnp.int32, (nb, n), 1)

    for t in range(_TOPK):
        x = work_ref[...]
        m = jnp.max(x, axis=1, keepdims=True)
        am = jnp.min(jnp.where(x == m, iota, n), axis=1, keepdims=True)
        idx_ref[:, t:t + 1] = am
        work_ref[...] = jnp.where(iota == am, -1.0, x)


def _prep_kernel(idx_ref, et16_ref, dr_ref, w_ref, seen_ref, wr_ref, sem):
    nb, ksel = idx_ref.shape
    nsel = nb * ksel
    nslots = dr_ref.shape[0]
    n_zero_row = seen_ref.shape[0] * 32
    wr_ref[...] = jnp.zeros_like(wr_ref)

    def clear_body(i, carry):
        seen_ref[i] = 0
        return carry

    jax.lax.fori_loop(0, seen_ref.shape[0], clear_body, 0)

    def body(j, carry):
        jb = jnp.minimum(j, nsel - 1)
        b = jb // ksel
        t = jb - b * ksel
        d = idx_ref[b, t]
        word = d // 32
        bit = d - word * 32
        seen = seen_ref[word]
        dup = (seen >> bit) & 1
        pad = j >= nsel
        skip = jnp.logical_or(dup == 1, pad)
        seen_ref[word] = jnp.where(pad, seen, seen | (1 << bit))
        dw = jnp.where(skip, n_zero_row, d)
        dr_ref[j] = jnp.where(skip, 0, d)
        pltpu.make_async_copy(et16_ref.at[pl.ds(dw, 1), :],
                              wr_ref.at[pl.ds(j, 1), :], sem).start()
        return carry

    jax.lax.fori_loop(0, nslots, body, 0)

    def wait_body(j, carry):
        pltpu.make_async_copy(et16_ref.at[pl.ds(0, 1), :],
                              wr_ref.at[pl.ds(0, 1), :], sem).wait()
        return carry

    jax.lax.fori_loop(0, nslots, wait_body, 0)

    lane = jax.lax.broadcasted_iota(jnp.int32, (16, 128), 1)
    row = jax.lax.broadcasted_iota(jnp.int32, (16, 128), 0)
    sel = jnp.where(lane // 16 == row, 1.0, 0.0).astype(jnp.float32)
    w_ref[...] = jnp.dot(wr_ref[...], sel,
                         preferred_element_type=jnp.float32)


def _sc_gather_kernel(dr_ref, w_ref, na_ref, out_ref, idr_v, w_v, rows_v,
                      acc_v, part_v, shared, sem):
    sid = lax.axis_index("s")
    base = sid * _SEL_PER_TILE
    d_o = rows_v.shape[1]
    nch = d_o // 16
    zero = jnp.zeros((16,), jnp.float32)

    pltpu.sync_copy(dr_ref.at[pl.ds(base, _SEL_PER_TILE)], idr_v)
    pltpu.sync_copy(w_ref.at[pl.ds(base, _SEL_PER_TILE), :], w_v)
    cr = pltpu.make_async_copy(na_ref.at[idr_v], rows_v, sem)
    cr.start()

    def zbody(b, carry):
        for c in range(nch):
            acc_v[b, pl.ds(c * 16, 16)] = zero
        return carry

    jax.lax.fori_loop(0, 8, zbody, 0)
    cr.wait()

    def jbody(j, carry):
        for b in range(8):
            wb = w_v[j, pl.ds(b * 16, 16)]
            for c in range(nch):
                sl = pl.ds(c * 16, 16)
                acc_v[b, sl] = acc_v[b, sl] + wb * rows_v[j, sl]
        return carry

    jax.lax.fori_loop(0, _SEL_PER_TILE, jbody, 0)

    pltpu.sync_copy(acc_v, shared.at[sid])
    plsc.subcore_barrier()

    @pl.when(sid == 0)
    def _():
        def tbody(t, carry):
            pltpu.sync_copy(shared.at[t], part_v)
            for b in range(8):
                for c in range(nch):
                    sl = pl.ds(c * 16, 16)
                    acc_v[b, sl] = acc_v[b, sl] + part_v[b, sl]
            return carry

        jax.lax.fori_loop(1, _NTILES, tbody, 0)
        pltpu.sync_copy(acc_v, out_ref)


def _final_kernel(oia_ref, sv_ref, o0_ref, o1_ref, o2_ref, out_ref):
    oia = oia_ref[...] * sv_ref[...]
    out_ref[:, 0:1] = jnp.sum(o0_ref[...] * oia, axis=1, keepdims=True)
    out_ref[:, 1:2] = jnp.sum(o1_ref[...] * oia, axis=1, keepdims=True)
    out_ref[:, 2:3] = jnp.sum(o2_ref[...] * oia, axis=1, keepdims=True)


def kernel(v, n_feats, aud, n_auds, ocr, n_ocrs, o, n_answ, temp_vid,
           temp_aud, temp_ocr):
    del aud, n_auds, ocr, n_ocrs, temp_aud, temp_ocr
    bq, d = v.shape
    n = n_feats.shape[0]

    e, et = pl.pallas_call(
        _score_kernel,
        grid=(n // _BLK,),
        in_specs=[pl.BlockSpec((bq, d), lambda i: (0, 0)),
                  pl.BlockSpec((_BLK, d), lambda i: (i, 0))],
        out_specs=(pl.BlockSpec((bq, _BLK), lambda i: (0, i)),
                   pl.BlockSpec((_BLK, bq), lambda i: (i, 0))),
        out_shape=(jax.ShapeDtypeStruct((bq, n), jnp.float32),
                   jax.ShapeDtypeStruct((n, bq), jnp.float32)),
    )(v, n_feats)

    idx, sv = pl.pallas_call(
        _topk_kernel,
        in_specs=[pl.BlockSpec(memory_space=pltpu.VMEM),
                  pl.BlockSpec(memory_space=pltpu.SMEM)],
        out_shape=(jax.ShapeDtypeStruct((bq, _TOPK), jnp.int32),
                   jax.ShapeDtypeStruct((bq, 1), jnp.float32)),
        scratch_shapes=[pltpu.VMEM((bq, n), jnp.float32)],
    )(e, temp_vid)

    et16 = jnp.pad(et, ((0, 1), (0, 16 - bq)))
    d_o = n_answ.shape[1]

    dr, w = pl.pallas_call(
        _prep_kernel,
        in_specs=[pl.BlockSpec(memory_space=pltpu.SMEM),
                  pl.BlockSpec(memory_space=pl.ANY)],
        out_specs=(pl.BlockSpec(memory_space=pltpu.SMEM),
                   pl.BlockSpec(memory_space=pltpu.VMEM)),
        out_shape=(jax.ShapeDtypeStruct((_NSEL_PAD,), jnp.int32),
                   jax.ShapeDtypeStruct((_NSEL_PAD, 128), jnp.float32)),
        scratch_shapes=[pltpu.SMEM((n // 32,), jnp.int32),
                        pltpu.VMEM((_NSEL_PAD, 16), jnp.float32),
                        pltpu.SemaphoreType.DMA],
    )(idx, et16)

    mesh = plsc.VectorSubcoreMesh(core_axis_name="c", subcore_axis_name="s")
    sc_call = pl.kernel(
        _sc_gather_kernel, mesh=mesh,
        out_type=jax.ShapeDtypeStruct((bq, d_o), jnp.float32),
        scratch_types=[
            pltpu.VMEM((_SEL_PER_TILE,), jnp.int32),
            pltpu.VMEM((_SEL_PER_TILE, 128), jnp.float32),
            pltpu.VMEM((_SEL_PER_TILE, d_o), jnp.float32),
            pltpu.VMEM((bq, d_o), jnp.float32),
            pltpu.VMEM((bq, d_o), jnp.float32),
            pltpu.VMEM_SHARED((_NTILES, bq, d_o), jnp.float32),
            pltpu.SemaphoreType.DMA,
        ],
    )
    oia = sc_call(dr, w, n_answ)

    o0, o1, o2 = o[:, 0, :], o[:, 1, :], o[:, 2, :]
    scores = pl.pallas_call(
        _final_kernel,
        out_shape=jax.ShapeDtypeStruct((bq, 3), jnp.float32),
    )(oia, sv, o0, o1, o2)
    return scores


# speedup vs baseline: 1.1744x; 1.1744x over previous
import jax
import jax.numpy as jnp
from jax import lax
from jax.experimental import pallas as pl
from jax.experimental.pallas import tpu as pltpu
from jax.experimental.pallas import tpu_sc as plsc

_TOPK = 25
_BLK = 2048
_NSEL_PAD = 256
_NTILES = 32
_SEL_PER_TILE = _NSEL_PAD // _NTILES


def _score_kernel(v_ref, nf_ref, e_ref, et_ref):
    v = v_ref[...]
    qn = v / jnp.maximum(jnp.sqrt(jnp.sum(v * v, axis=1, keepdims=True)), 1e-12)
    k = nf_ref[...]
    kn = k / jnp.maximum(jnp.sqrt(jnp.sum(k * k, axis=1, keepdims=True)), 1e-12)
    s = jax.lax.dot_general(qn, kn, (((1,), (1,)), ((), ())),
                            preferred_element_type=jnp.float32)
    e_ref[...] = jnp.exp(jnp.clip(s, 0.0, 1.0))
    st = jax.lax.dot_general(kn, qn, (((1,), (1,)), ((), ())),
                             preferred_element_type=jnp.float32)
    et_ref[...] = jnp.exp(jnp.clip(st, 0.0, 1.0))


def _topk_kernel(e_ref, tv_ref, idx_ref, sc_ref, work_ref):
    work_ref[...] = e_ref[...]
    nb, n = work_ref.shape
    gate = 2.0 * jax.nn.sigmoid(tv_ref[0])
    den = jnp.sum(e_ref[...], axis=1, keepdims=True)
    sc_ref[...] = gate / den
    iota = jax.lax.broadcasted_iota(jnp.int32, (nb, n), 1)

    for t in range(_TOPK):
        x = work_ref[...]
        m = jnp.max(x, axis=1, keepdims=True)
        am = jnp.min(jnp.where(x == m, iota, n), axis=1, keepdims=True)
        idx_ref[:, t:t + 1] = am
        work_ref[...] = jnp.where(iota == am, -1.0, x)


def _prep_kernel(idx_ref, et16_ref, dr_ref, w_ref, seen_ref, wr_ref, sem):
    nb, ksel = idx_ref.shape
    nsel = nb * ksel
    nslots = dr_ref.shape[0]
    n_zero_row = seen_ref.shape[0] * 32
    wr_ref[...] = jnp.zeros_like(wr_ref)

    def clear_body(i, carry):
        seen_ref[i] = 0
        return carry

    jax.lax.fori_loop(0, seen_ref.shape[0], clear_body, 0)

    def body(j, carry):
        jb = jnp.minimum(j, nsel - 1)
        b = jb // ksel
        t = jb - b * ksel
        d = idx_ref[b, t]
        word = d // 32
        bit = d - word * 32
        seen = seen_ref[word]
        dup = (seen >> bit) & 1
        pad = j >= nsel
        skip = jnp.logical_or(dup == 1, pad)
        seen_ref[word] = jnp.where(pad, seen, seen | (1 << bit))
        dw = jnp.where(skip, n_zero_row, d)
        dr_ref[j] = jnp.where(skip, 0, d)
        pltpu.make_async_copy(et16_ref.at[pl.ds(dw, 1), :],
                              wr_ref.at[pl.ds(j, 1), :], sem).start()
        return carry

    jax.lax.fori_loop(0, nslots, body, 0)

    def wait_body(j, carry):
        pltpu.make_async_copy(et16_ref.at[pl.ds(0, 1), :],
                              wr_ref.at[pl.ds(0, 1), :], sem).wait()
        return carry

    jax.lax.fori_loop(0, nslots, wait_body, 0)

    lane = jax.lax.broadcasted_iota(jnp.int32, (16, 128), 1)
    row = jax.lax.broadcasted_iota(jnp.int32, (16, 128), 0)
    sel = jnp.where(lane // 16 == row, 1.0, 0.0).astype(jnp.float32)
    w_ref[...] = jnp.dot(wr_ref[...], sel,
                         preferred_element_type=jnp.float32)


def _sc_gather_kernel(dr_ref, w_ref, na_ref, out_ref, idr_v, w_v, rows_v,
                      acc_v, sem):
    wid = lax.axis_index("s") * 2 + lax.axis_index("c")
    base = wid * _SEL_PER_TILE
    d_o = rows_v.shape[1]
    nch = d_o // 16
    zero = jnp.zeros((16,), jnp.float32)

    pltpu.sync_copy(dr_ref.at[pl.ds(base, _SEL_PER_TILE)], idr_v)
    pltpu.sync_copy(w_ref.at[pl.ds(base, _SEL_PER_TILE), :], w_v)
    cr = pltpu.make_async_copy(na_ref.at[idr_v], rows_v, sem)
    cr.start()

    def zbody(b, carry):
        for c in range(nch):
            acc_v[b, pl.ds(c * 16, 16)] = zero
        return carry

    jax.lax.fori_loop(0, 8, zbody, 0)
    cr.wait()

    def jbody(j, carry):
        for b in range(8):
            wb = w_v[j, pl.ds(b * 16, 16)]
            for c in range(nch):
                sl = pl.ds(c * 16, 16)
                acc_v[b, sl] = acc_v[b, sl] + wb * rows_v[j, sl]
        return carry

    jax.lax.fori_loop(0, _SEL_PER_TILE, jbody, 0)
    pltpu.sync_copy(acc_v, out_ref.at[wid])


def _final_kernel(parts_ref, sv_ref, o0_ref, o1_ref, o2_ref, out_ref):
    oia = jnp.sum(parts_ref[...], axis=0) * sv_ref[...]
    out_ref[:, 0:1] = jnp.sum(o0_ref[...] * oia, axis=1, keepdims=True)
    out_ref[:, 1:2] = jnp.sum(o1_ref[...] * oia, axis=1, keepdims=True)
    out_ref[:, 2:3] = jnp.sum(o2_ref[...] * oia, axis=1, keepdims=True)


def kernel(v, n_feats, aud, n_auds, ocr, n_ocrs, o, n_answ, temp_vid,
           temp_aud, temp_ocr):
    del aud, n_auds, ocr, n_ocrs, temp_aud, temp_ocr
    bq, d = v.shape
    n = n_feats.shape[0]

    e, et = pl.pallas_call(
        _score_kernel,
        grid=(n // _BLK,),
        in_specs=[pl.BlockSpec((bq, d), lambda i: (0, 0)),
                  pl.BlockSpec((_BLK, d), lambda i: (i, 0))],
        out_specs=(pl.BlockSpec((bq, _BLK), lambda i: (0, i)),
                   pl.BlockSpec((_BLK, bq), lambda i: (i, 0))),
        out_shape=(jax.ShapeDtypeStruct((bq, n), jnp.float32),
                   jax.ShapeDtypeStruct((n, bq), jnp.float32)),
    )(v, n_feats)

    idx, sv = pl.pallas_call(
        _topk_kernel,
        in_specs=[pl.BlockSpec(memory_space=pltpu.VMEM),
                  pl.BlockSpec(memory_space=pltpu.SMEM)],
        out_shape=(jax.ShapeDtypeStruct((bq, _TOPK), jnp.int32),
                   jax.ShapeDtypeStruct((bq, 1), jnp.float32)),
        scratch_shapes=[pltpu.VMEM((bq, n), jnp.float32)],
    )(e, temp_vid)

    et16 = jnp.pad(et, ((0, 1), (0, 16 - bq)))
    d_o = n_answ.shape[1]

    dr, w = pl.pallas_call(
        _prep_kernel,
        in_specs=[pl.BlockSpec(memory_space=pltpu.SMEM),
                  pl.BlockSpec(memory_space=pl.ANY)],
        out_specs=(pl.BlockSpec(memory_space=pltpu.SMEM),
                   pl.BlockSpec(memory_space=pltpu.VMEM)),
        out_shape=(jax.ShapeDtypeStruct((_NSEL_PAD,), jnp.int32),
                   jax.ShapeDtypeStruct((_NSEL_PAD, 128), jnp.float32)),
        scratch_shapes=[pltpu.SMEM((n // 32,), jnp.int32),
                        pltpu.VMEM((_NSEL_PAD, 16), jnp.float32),
                        pltpu.SemaphoreType.DMA],
    )(idx, et16)

    mesh = plsc.VectorSubcoreMesh(core_axis_name="c", subcore_axis_name="s")
    sc_call = pl.kernel(
        _sc_gather_kernel, mesh=mesh,
        out_type=jax.ShapeDtypeStruct((_NTILES, bq, d_o), jnp.float32),
        scratch_types=[
            pltpu.VMEM((_SEL_PER_TILE,), jnp.int32),
            pltpu.VMEM((_SEL_PER_TILE, 128), jnp.float32),
            pltpu.VMEM((_SEL_PER_TILE, d_o), jnp.float32),
            pltpu.VMEM((bq, d_o), jnp.float32),
            pltpu.SemaphoreType.DMA,
        ],
    )
    oia = sc_call(dr, w, n_answ)

    o0, o1, o2 = o[:, 0, :], o[:, 1, :], o[:, 2, :]
    scores = pl.pallas_call(
        _final_kernel,
        out_shape=jax.ShapeDtypeStruct((bq, 3), jnp.float32),
    )(oia, sv, o0, o1, o2)
    return scores
